# baseline (speedup 1.0000x reference)
import functools

import jax
import jax.numpy as jnp
from jax import lax
from jax.experimental import pallas as pl
from jax.experimental.pallas import tpu as pltpu
from jax.experimental.pallas import tpu_sc as plsc

NUM_CORES = 2
NUM_SUBCORES = 16
LANES = 16
NW = NUM_CORES * NUM_SUBCORES
CHUNKS = (504, 440, 80)


def _build_sc_call(seq: int, table_rows: int, hid: int):
    max_len = (table_rows + 1) // 2
    rows_per_w = seq // NW
    assert sum(CHUNKS) == seq
    j_offs = [sum(CHUNKS[:k]) for k in range(len(CHUNKS))]
    wins = [((rows_per_w + cw - 1 + 7) // 8) * 8 for cw in CHUNKS]
    buf_rows = [max(wins[0::2]), max(wins[1::2])]
    idx_rows = [((r + LANES - 1) // LANES) * LANES for r in buf_rows]

    mesh = plsc.VectorSubcoreMesh(
        core_axis_name="c", subcore_axis_name="s",
        num_cores=NUM_CORES, num_subcores=NUM_SUBCORES)

    @functools.partial(
        pl.kernel,
        out_type=jax.ShapeDtypeStruct((seq, seq, hid), jnp.float32),
        mesh=mesh,
        scratch_types=[
            pltpu.VMEM((idx_rows[0],), jnp.int32),
            pltpu.VMEM((idx_rows[1],), jnp.int32),
            pltpu.VMEM((buf_rows[0], hid), jnp.float32),
            pltpu.VMEM((buf_rows[1], hid), jnp.float32),
            pltpu.SemaphoreType.DMA,
            pltpu.SemaphoreType.DMA,
            pltpu.SemaphoreType.DMA,
            pltpu.SemaphoreType.DMA,
        ],
    )
    def sc_gather(table_hbm, out_hbm, idx0, idx1, wb0, wb1, gs0, gs1, ws0, ws1):
        idx, wbuf, gsem, wsem = (idx0, idx1), (wb0, wb1), (gs0, gs1), (ws0, ws1)
        wid = lax.axis_index("s") * NUM_CORES + lax.axis_index("c")
        lane = lax.iota(jnp.int32, LANES)
        i0 = wid * rows_per_w

        def gather_window(b):
            p = b % 2
            w = wins[b]
            hi = i0 - j_offs[b] + (max_len - 1) + (rows_per_w - 1)
            for g in range((w + LANES - 1) // LANES):
                idx[p][pl.ds(g * LANES, LANES)] = jnp.maximum(
                    (hi - g * LANES) - lane, 0)
            return pltpu.make_async_copy(
                table_hbm.at[idx[p].at[pl.ds(0, w)]],
                wbuf[p].at[pl.ds(0, w), :],
                gsem[p])

        def row_copy(b, di):
            p = b % 2
            return pltpu.make_async_copy(
                wbuf[p].at[pl.ds(rows_per_w - 1 - di, CHUNKS[b]), :],
                out_hbm.at[i0 + di, pl.ds(j_offs[b], CHUNKS[b]), :],
                wsem[p])

        def fire(b):
            for di in range(rows_per_w):
                row_copy(b, di).start()

        def drain(b):
            for di in range(rows_per_w):
                row_copy(b, di).wait()

        g = gather_window(0)
        g.start()
        g.wait()
        fire(0)
        g = gather_window(1)
        g.start()
        g.wait()
        fire(1)
        drain(0)
        g = gather_window(2)
        g.start()
        g.wait()
        fire(2)
        drain(1)
        drain(2)

    return sc_gather


def kernel(x, relative_positions):
    seq = x.shape[1]
    table_rows, hid = relative_positions.shape
    call = _build_sc_call(seq, table_rows, hid)
    return call(relative_positions)


# speedup vs baseline: 1.0347x; 1.0347x over previous
import functools

import jax
import jax.numpy as jnp
from jax import lax
from jax.experimental import pallas as pl
from jax.experimental.pallas import tpu as pltpu
from jax.experimental.pallas import tpu_sc as plsc

NUM_CORES = 2
NUM_SUBCORES = 16
LANES = 16
NW = NUM_CORES * NUM_SUBCORES
SPLIT = 624


def _build_sc_call(seq: int, table_rows: int, hid: int):
    max_len = (table_rows + 1) // 2
    rows_per_w = seq // NW
    rows_per_sc = rows_per_w * NUM_SUBCORES
    rest = seq - SPLIT

    win_t = ((rows_per_w + SPLIT - 1 + 7) // 8) * 8
    idx_t = ((win_t + LANES - 1) // LANES) * LANES
    win_s = ((rows_per_sc + seq - 1 + NUM_SUBCORES * LANES - 1)
             // (NUM_SUBCORES * LANES)) * (NUM_SUBCORES * LANES)
    shard = win_s // NUM_SUBCORES

    mesh = plsc.VectorSubcoreMesh(
        core_axis_name="c", subcore_axis_name="s",
        num_cores=NUM_CORES, num_subcores=NUM_SUBCORES)

    @functools.partial(
        pl.kernel,
        out_type=jax.ShapeDtypeStruct((seq, seq, hid), jnp.float32),
        mesh=mesh,
        scratch_types=[
            pltpu.VMEM((idx_t,), jnp.int32),
            pltpu.VMEM((win_t, hid), jnp.float32),
            pltpu.VMEM((shard, hid), jnp.float32),
            pltpu.VMEM_SHARED((win_s, hid), jnp.float32),
            pltpu.SemaphoreType.DMA,
            pltpu.SemaphoreType.DMA,
            pltpu.SemaphoreType.DMA,
        ],
    )
    def sc_gather(table_hbm, out_hbm, idx, wbuf, tbuf, shared, gsem, wsem, ssem):
        c = lax.axis_index("c")
        s = lax.axis_index("s")
        lane = lax.iota(jnp.int32, LANES)
        sc_i0 = c * rows_per_sc
        i0 = sc_i0 + s * rows_per_w

        hi_s = sc_i0 + (rows_per_sc - 1) + (max_len - 1)
        off = s * shard
        for g in range(shard // LANES):
            idx[pl.ds(g * LANES, LANES)] = jnp.maximum(
                (hi_s - off - g * LANES) - lane, 0)
        g1 = pltpu.make_async_copy(
            table_hbm.at[idx.at[pl.ds(0, shard)]], tbuf, gsem)
        g1.start()
        g1.wait()
        pltpu.sync_copy(tbuf, shared.at[pl.ds(off, shard), :])

        hi_t = i0 + (rows_per_w - 1) + (max_len - 1)
        for g in range(win_t // LANES):
            idx[pl.ds(g * LANES, LANES)] = jnp.maximum(
                (hi_t - g * LANES) - lane, 0)
        g2 = pltpu.make_async_copy(
            table_hbm.at[idx.at[pl.ds(0, win_t)]], wbuf, gsem)
        g2.start()
        g2.wait()

        plsc.subcore_barrier()

        def stream_copy(di):
            return pltpu.make_async_copy(
                wbuf.at[pl.ds(rows_per_w - 1 - di, SPLIT), :],
                out_hbm.at[i0 + di, pl.ds(0, SPLIT), :],
                wsem)

        def spmem_copy(di):
            dd = (i0 - sc_i0) + di
            return pltpu.make_async_copy(
                shared.at[pl.ds((rows_per_sc - 1) - dd + SPLIT, rest), :],
                out_hbm.at[i0 + di, pl.ds(SPLIT, rest), :],
                ssem)

        for di in range(rows_per_w):
            stream_copy(di).start()
            spmem_copy(di).start()
        for di in range(rows_per_w):
            stream_copy(di).wait()
            spmem_copy(di).wait()

    return sc_gather


def kernel(x, relative_positions):
    seq = x.shape[1]
    table_rows, hid = relative_positions.shape
    call = _build_sc_call(seq, table_rows, hid)
    return call(relative_positions)
